# baseline (speedup 1.0000x reference)
import functools

import jax
import jax.numpy as jnp
from jax import lax
from jax.experimental import pallas as pl
from jax.experimental.pallas import tpu as pltpu
from jax.experimental.pallas import tpu_sc as plsc

B = 16384
CONT = 13
NF = 26
V = 100000
D = 16

NC = 2
NS = 16
NW = NC * NS
ROWS_W = B // NW
RP = B // 128
WP = ROWS_W // 128
CB = 64
NCHUNK = ROWS_W // CB
OUT_W = CONT + NF * D


def _sc_kernel(cate_hbm, cont_hbm, table_hbm, out_hbm,
               fcate_v, em_v, cont_v, row_v, gsem):
    wid = lax.axis_index("s") * NC + lax.axis_index("c")

    for f in range(NF):
        pltpu.sync_copy(cate_hbm.at[pl.ds(f * RP + wid * WP, WP)],
                        fcate_v.at[f])
    def fire(c, buf):
        p = c // 2
        s = pl.ds(lax.rem(c, 2) * CB, CB)
        for f in range(NF):
            pltpu.async_copy(table_hbm.at[f].at[fcate_v.at[f, p, s]],
                             em_v.at[buf, f], gsem)

    def drain(c, buf):
        p = c // 2
        s = pl.ds(lax.rem(c, 2) * CB, CB)
        for f in range(NF):
            pltpu.make_async_copy(table_hbm.at[f].at[fcate_v.at[f, p, s]],
                                  em_v.at[buf, f], gsem).wait()

    fire(0, 0)

    def chunk_body(c, carry):
        buf = lax.rem(c, 2)
        row0 = wid * ROWS_W + c * CB
        pltpu.sync_copy(cont_hbm.at[pl.ds(row0, CB)], cont_v)
        drain(c, buf)

        @pl.when(c + 1 < NCHUNK)
        def _():
            fire(c + 1, lax.rem(c + 1, 2))

        def assemble(b, cc):
            row_v[b, pl.ds(0, 16)] = cont_v[b]
            for f in range(NF):
                row_v[b, pl.ds(CONT + f * D, D)] = em_v[buf, f, b]
            return cc

        lax.fori_loop(0, CB, assemble, c)
        pltpu.sync_copy(row_v, out_hbm.at[pl.ds(row0, CB)])
        return carry

    lax.fori_loop(0, NCHUNK, chunk_body, 0)


@jax.jit
def kernel(x_cont, x_cate, tables):
    cate_t = x_cate.T.reshape(NF * RP, 128)
    cont_pad = jnp.pad(x_cont, ((0, 0), (0, 3)))
    mesh = plsc.VectorSubcoreMesh(core_axis_name="c", subcore_axis_name="s")
    run = functools.partial(
        pl.kernel,
        mesh=mesh,
        compiler_params=pltpu.CompilerParams(use_tc_tiling_on_sc=False),
        out_type=jax.ShapeDtypeStruct((B, OUT_W), jnp.float32),
        scratch_types=[
            pltpu.VMEM((NF, WP, 128), jnp.int32),
            pltpu.VMEM((2, NF, CB, D), jnp.float32),
            pltpu.VMEM((CB, 16), jnp.float32),
            pltpu.VMEM((CB, OUT_W), jnp.float32),
            pltpu.SemaphoreType.DMA,
        ],
    )(_sc_kernel)
    return run(cate_t, cont_pad, tables)


# speedup vs baseline: 2.0643x; 2.0643x over previous
import functools

import jax
import jax.numpy as jnp
from jax import lax
from jax.experimental import pallas as pl
from jax.experimental.pallas import tpu as pltpu
from jax.experimental.pallas import tpu_sc as plsc

B = 16384
CONT = 13
NF = 26
V = 100000
D = 16

NC = 2
NS = 16
NW = NC * NS
ROWS_W = B // NW
RP = B // 128
WP = ROWS_W // 128
CB = 128
NCHUNK = ROWS_W // CB
FD = NF * D
OUT_W = CONT + FD


def _sc_kernel(cate_hbm, cont_hbm, table_hbm, out_hbm,
               fcate_v, gbuf_v, gsem, wsem):
    wid = lax.axis_index("s") * NC + lax.axis_index("c")
    base = wid * ROWS_W

    for f in range(NF):
        pltpu.sync_copy(cate_hbm.at[pl.ds(f * RP + wid * WP, WP)],
                        fcate_v.at[f])

    def fire(c, buf):
        def g_body(g, carry):
            pltpu.async_copy(
                table_hbm.at[g].at[fcate_v.at[g // D, c]],
                gbuf_v.at[buf, g], gsem)
            return carry
        lax.fori_loop(0, FD, g_body, 0)

    def drain(c, buf):
        def g_body(g, carry):
            pltpu.make_async_copy(
                table_hbm.at[g].at[fcate_v.at[g // D, c]],
                gbuf_v.at[buf, g], gsem).wait()
            return carry
        lax.fori_loop(0, FD, g_body, 0)

    fire(0, 0)

    def chunk_body(c, carry):
        buf = lax.rem(c, 2)
        drain(c, buf)

        @pl.when(c + 1 < NCHUNK)
        def _():
            fire(c + 1, lax.rem(c + 1, 2))

        def w_body(g, carry2):
            pltpu.async_copy(
                gbuf_v.at[buf, g],
                out_hbm.at[CONT + g, pl.ds(base + c * CB, CB)], wsem)
            return carry2

        lax.fori_loop(0, FD, w_body, 0)

        def wd_body(g, carry2):
            pltpu.make_async_copy(
                gbuf_v.at[buf, g],
                out_hbm.at[CONT + g, pl.ds(base + c * CB, CB)], wsem).wait()
            return carry2

        lax.fori_loop(0, FD, wd_body, 0)
        return carry

    lax.fori_loop(0, NCHUNK, chunk_body, 0)

    def cont_body(k, carry):
        pltpu.sync_copy(cont_hbm.at[k, pl.ds(base, ROWS_W)],
                        out_hbm.at[k, pl.ds(base, ROWS_W)])
        return carry

    lax.fori_loop(0, CONT, cont_body, 0)


@jax.jit
def kernel(x_cont, x_cate, tables):
    cate_t = x_cate.T.reshape(NF * RP, 128)
    cont_t = x_cont.T
    table_t = tables.transpose(0, 2, 1).reshape(NF * D, V)
    mesh = plsc.VectorSubcoreMesh(core_axis_name="c", subcore_axis_name="s")
    run = functools.partial(
        pl.kernel,
        mesh=mesh,
        compiler_params=pltpu.CompilerParams(use_tc_tiling_on_sc=False),
        out_type=jax.ShapeDtypeStruct((OUT_W, B), jnp.float32),
        scratch_types=[
            pltpu.VMEM((NF, WP, 128), jnp.int32),
            pltpu.VMEM((2, FD, CB), jnp.float32),
            pltpu.SemaphoreType.DMA,
            pltpu.SemaphoreType.DMA,
        ],
    )(_sc_kernel)
    yt = run(cate_t, cont_t, table_t)
    return yt.T


# speedup vs baseline: 2.2294x; 1.0800x over previous
import functools

import jax
import jax.numpy as jnp
from jax import lax
from jax.experimental import pallas as pl
from jax.experimental.pallas import tpu as pltpu
from jax.experimental.pallas import tpu_sc as plsc

B = 16384
CONT = 13
NF = 26
V = 100000
D = 16

NC = 2
NS = 16
NW = NC * NS
ROWS_W = B // NW
RP = B // 128
WP = ROWS_W // 128
CB = 128
NCHUNK = ROWS_W // CB
FD = NF * D
OUT_W = CONT + FD


def _sc_kernel(cate_hbm, cont_hbm, table_hbm, out_hbm,
               fcate_v, gbuf_v, gsem, wsem, csem):
    wid = lax.axis_index("s") * NC + lax.axis_index("c")
    base = wid * ROWS_W

    for k in range(CONT):
        pltpu.async_copy(cont_hbm.at[k, pl.ds(base, ROWS_W)],
                         out_hbm.at[k, pl.ds(base, ROWS_W)], csem)

    idescs = []
    for f in range(NF):
        idescs.append(pltpu.async_copy(
            cate_hbm.at[pl.ds(f * RP + wid * WP, WP)], fcate_v.at[f], wsem))
    for dsc in idescs:
        dsc.wait()

    def fire(c, buf):
        def g_body(f, carry):
            for d in range(D):
                pltpu.async_copy(
                    table_hbm.at[f * D + d].at[fcate_v.at[f, c]],
                    gbuf_v.at[buf, f * D + d], gsem)
            return carry
        lax.fori_loop(0, NF, g_body, 0)

    def drain(c, buf):
        def g_body(f, carry):
            for d in range(D):
                pltpu.make_async_copy(
                    table_hbm.at[f * D + d].at[fcate_v.at[f, c]],
                    gbuf_v.at[buf, f * D + d], gsem).wait()
            return carry
        lax.fori_loop(0, NF, g_body, 0)

    fire(0, 0)

    def chunk_body(c, carry):
        buf = lax.rem(c, 2)
        drain(c, buf)

        @pl.when(c + 1 < NCHUNK)
        def _():
            fire(c + 1, lax.rem(c + 1, 2))

        def w_body(f, carry2):
            for d in range(D):
                pltpu.async_copy(
                    gbuf_v.at[buf, f * D + d],
                    out_hbm.at[CONT + f * D + d, pl.ds(base + c * CB, CB)],
                    wsem)
            return carry2

        lax.fori_loop(0, NF, w_body, 0)

        def wd_body(f, carry2):
            for d in range(D):
                pltpu.make_async_copy(
                    gbuf_v.at[buf, f * D + d],
                    out_hbm.at[CONT + f * D + d, pl.ds(base + c * CB, CB)],
                    wsem).wait()
            return carry2

        lax.fori_loop(0, NF, wd_body, 0)
        return carry

    lax.fori_loop(0, NCHUNK, chunk_body, 0)

    for k in range(CONT):
        pltpu.make_async_copy(cont_hbm.at[k, pl.ds(base, ROWS_W)],
                              out_hbm.at[k, pl.ds(base, ROWS_W)], csem).wait()


@jax.jit
def kernel(x_cont, x_cate, tables):
    cate_t = x_cate.T.reshape(NF * RP, 128)
    cont_t = x_cont.T
    table_t = tables.transpose(0, 2, 1).reshape(NF * D, V)
    mesh = plsc.VectorSubcoreMesh(core_axis_name="c", subcore_axis_name="s")
    run = functools.partial(
        pl.kernel,
        mesh=mesh,
        compiler_params=pltpu.CompilerParams(use_tc_tiling_on_sc=False),
        out_type=jax.ShapeDtypeStruct((OUT_W, B), jnp.float32),
        scratch_types=[
            pltpu.VMEM((NF, WP, 128), jnp.int32),
            pltpu.VMEM((2, FD, CB), jnp.float32),
            pltpu.SemaphoreType.DMA,
            pltpu.SemaphoreType.DMA,
            pltpu.SemaphoreType.DMA,
        ],
    )(_sc_kernel)
    yt = run(cate_t, cont_t, table_t)
    return yt.T
